# baseline (speedup 1.0000x reference)
import jax
import jax.numpy as jnp
from jax import lax
from jax.experimental import pallas as pl
from jax.experimental.pallas import tpu as pltpu
from jax.experimental.pallas import tpu_sc as plsc

NROWS = 1000000
D = 64
B = 16384
NC = 2
NS = 16
NW = NC * NS
BPW = B // NW
L = 16
CHUNK = 128
NCHUNK = BPW // CHUNK
G = CHUNK // L


def _body(users_hbm, pos_hbm, neg_hbm, utab_hbm, itab_hbm,
          pos_out, neg_out,
          uidx, pidx, nidx,
          ul0, pl0, nl0, ul1, pl1, nl1,
          ur0, pr0, nr0, ur1, pr1, nr1,
          psc, nsc,
          su0, sp0, sn0, su1, sp1, sn1):
    wid = lax.axis_index("s") * NC + lax.axis_index("c")
    base = wid * BPW

    pltpu.sync_copy(users_hbm.at[pl.ds(base, BPW)], uidx)
    pltpu.sync_copy(pos_hbm.at[pl.ds(base, BPW)], pidx)
    pltpu.sync_copy(neg_hbm.at[pl.ds(base, BPW)], nidx)

    lists = ((ul0, pl0, nl0), (ul1, pl1, nl1))
    rows = ((ur0, pr0, nr0), (ur1, pr1, nr1))
    sems = ((su0, sp0, sn0), (su1, sp1, sn1))

    def build(c):
        ul, plst, nl = lists[c % 2]
        off = c * CHUNK

        def grp(g, carry):
            ru = uidx[pl.ds(off + g * L, L)]
            rp = pidx[pl.ds(off + g * L, L)]
            rn = nidx[pl.ds(off + g * L, L)]
            for d in range(D):
                ul[pl.ds(d * CHUNK + g * L, L)] = ru + d * NROWS
                plst[pl.ds(d * CHUNK + g * L, L)] = rp + d * NROWS
                nl[pl.ds(d * CHUNK + g * L, L)] = rn + d * NROWS
            return carry

        lax.fori_loop(0, G, grp, 0, unroll=False)

    def start(c):
        ul, plst, nl = lists[c % 2]
        ur, pr, nr = rows[c % 2]
        su, sp, sn = sems[c % 2]
        return (pltpu.async_copy(utab_hbm.at[ul], ur, su),
                pltpu.async_copy(itab_hbm.at[plst], pr, sp),
                pltpu.async_copy(itab_hbm.at[nl], nr, sn))

    def compute(c, copies):
        for cp in copies:
            cp.wait()
        ur, pr, nr = rows[c % 2]
        off = c * CHUNK

        def grp(g, carry):
            accp = jnp.zeros((L,), jnp.float32)
            accn = jnp.zeros((L,), jnp.float32)

            def dstep(d, accs):
                ap, an = accs
                u = ur[pl.ds(d * CHUNK + g * L, L)]
                ap = ap + u * pr[pl.ds(d * CHUNK + g * L, L)]
                an = an + u * nr[pl.ds(d * CHUNK + g * L, L)]
                return (ap, an)

            accp, accn = lax.fori_loop(0, D, dstep, (accp, accn), unroll=8)
            psc[pl.ds(off + g * L, L)] = accp
            nsc[pl.ds(off + g * L, L)] = accn
            return carry

        lax.fori_loop(0, G, grp, 0, unroll=False)

    build(0)
    inflight = start(0)
    for c in range(NCHUNK):
        if c + 1 < NCHUNK:
            build(c + 1)
            nxt = start(c + 1)
        else:
            nxt = None
        compute(c, inflight)
        inflight = nxt

    pltpu.sync_copy(psc, pos_out.at[pl.ds(base, BPW)])
    pltpu.sync_copy(nsc, neg_out.at[pl.ds(base, BPW)])


@jax.jit
def kernel(users, pos_items, neg_items, user_table, item_table):
    mesh = plsc.VectorSubcoreMesh(core_axis_name="c", subcore_axis_name="s",
                                  num_cores=NC, num_subcores=NS)
    k = pl.kernel(
        _body,
        out_type=(jax.ShapeDtypeStruct((B,), jnp.float32),
                  jax.ShapeDtypeStruct((B,), jnp.float32)),
        mesh=mesh,
        scratch_types=(
            [pltpu.VMEM((BPW,), jnp.int32)] * 3
            + [pltpu.VMEM((CHUNK * D,), jnp.int32)] * 6
            + [pltpu.VMEM((CHUNK * D,), jnp.float32)] * 6
            + [pltpu.VMEM((BPW,), jnp.float32)] * 2
            + [pltpu.SemaphoreType.DMA] * 6
        ),
        compiler_params=pltpu.CompilerParams(needs_layout_passes=False,
                                             use_tc_tiling_on_sc=False),
        name="bprmf_sc_score",
    )
    ut = user_table.T.reshape(NROWS * D)
    it = item_table.T.reshape(NROWS * D)
    return k(users, pos_items, neg_items, ut, it)


# speedup vs baseline: 4.4730x; 4.4730x over previous
import functools

import jax
import jax.numpy as jnp
from jax import lax
from jax.experimental import pallas as pl
from jax.experimental.pallas import tpu as pltpu
from jax.experimental.pallas import tpu_sc as plsc

NROWS = 1000000
D = 64
B = 16384
NC = 2
NS = 16
NW = NC * NS
BPW = B // NW
L = 16
SW = 384
FULL_SLABS = 2604
TAIL_LO = FULL_SLABS * SW
TRASH_U = B
TRASH_I = 2 * B
UCAP = B + 16
ICAP = 2 * B + 16
CB = 8192
EMBW = 128
HB = BPW // 2


def _sweep_body(users_hbm, pos_hbm, neg_hbm, utab_hbm, itab_hbm,
                tailu_hbm, taili_hbm,
                u_emb, i_emb,
                cbuf, ulist, ilist, slab, tailblk,
                stg_r, stg_b, outbuf, outc, blist,
                osem):
    wid = lax.axis_index("s") * NC + lax.axis_index("c")
    nsl = jnp.where(wid < 12, 82, 81)
    lo = (wid * 81 + jnp.minimum(wid, 12)) * SW
    span = nsl * SW
    hi = jnp.where(wid == 31, NROWS, lo + span)

    iota = lax.iota(jnp.int32, L)
    tr65 = iota * 65

    def scan_array(arr, lst, b_off, cnt0):
        def chunk(ci, cnt):
            pltpu.sync_copy(arr.at[pl.ds(ci * CB, CB)], cbuf)

            def grp(g, c):
                r = cbuf[pl.ds(g * L, L)]
                m = (r >= lo) & (r < hi)
                b = ci * CB + g * L + b_off + iota
                h = (b << 15) | (r - lo)
                plsc.store_compressed(lst.at[pl.ds(c, L)], h, mask=m)
                return c + plsc.all_reduce_population_count(m)[0]

            return lax.fori_loop(0, CB // L, grp, cnt)

        return lax.fori_loop(0, B // CB, chunk, cnt0)

    ucnt = scan_array(users_hbm, ulist, 0, 0)
    ulist[pl.ds(ucnt, L)] = jnp.full((L,), 32767, jnp.int32)
    icnt = scan_array(pos_hbm, ilist, 0, 0)
    icnt = scan_array(neg_hbm, ilist, B, icnt)
    ilist[pl.ds(icnt, L)] = jnp.full((L,), 32767, jnp.int32)

    def flush16(rr, bb, emb, row_major_tail=False):
        pltpu.make_async_copy(outc, emb.at[blist], osem).wait()
        for d in range(D):
            dv = jnp.full((L,), d, jnp.int32)
            if row_major_tail:
                v = plsc.load_gather(tailblk, [rr, dv])
            else:
                v = plsc.load_gather(slab, [dv, rr])
            plsc.store_scatter(outbuf, [tr65 + d], v)
        for k in range(L):
            for q in range(D // L):
                outc[k, pl.ds(q * L, L)] = outbuf[pl.ds(k * 65 + q * L, L)]
        blist[pl.ds(0, L)] = bb
        pltpu.async_copy(outc, emb.at[blist], osem)

    def run_sweep(tab, tail_hbm, lst, lcnt, emb, trash):
        blist[pl.ds(0, L)] = jnp.full((L,), trash, jnp.int32)
        pltpu.async_copy(outc, emb.at[blist], osem)

        def scan_slab(s_lo, s_w, row_major_tail=False):
            ngrp = (lcnt + L - 1) // L

            def grp(g, scnt):
                h = lst[pl.ds(g * L, L)]
                rl = h & 32767
                bb0 = h >> 15
                m = (rl >= s_lo) & (rl < s_lo + s_w)
                plsc.store_compressed(stg_r.at[pl.ds(scnt, L)], rl - s_lo,
                                      mask=m)
                plsc.store_compressed(stg_b.at[pl.ds(scnt, L)], bb0, mask=m)
                snew = scnt + plsc.all_reduce_population_count(m)[0]

                @pl.when(snew >= L)
                def _():
                    flush16(stg_r[pl.ds(0, L)], stg_b[pl.ds(0, L)], emb,
                            row_major_tail)
                    stg_r[pl.ds(0, L)] = stg_r[pl.ds(L, L)]
                    stg_b[pl.ds(0, L)] = stg_b[pl.ds(L, L)]

                return jnp.where(snew >= L, snew - L, snew)

            scnt_end = lax.fori_loop(0, ngrp, grp, 0)

            @pl.when(scnt_end > 0)
            def _():
                sel = iota < scnt_end
                rr = jnp.where(sel, stg_r[pl.ds(0, L)], 0)
                bb = jnp.where(sel, stg_b[pl.ds(0, L)],
                               jnp.full((L,), trash, jnp.int32))
                flush16(rr, bb, emb, row_major_tail)

        def slab_iter(s, c):
            pltpu.sync_copy(tab.at[:, pl.ds(lo + s * SW, SW)], slab)
            scan_slab(s * SW, SW)
            return c

        lax.fori_loop(0, nsl, slab_iter, 0)

        @pl.when(wid == 31)
        def _():
            pltpu.sync_copy(tail_hbm, tailblk)
            scan_slab(81 * SW, 64, row_major_tail=True)

        pltpu.make_async_copy(outc, emb.at[blist], osem).wait()

    run_sweep(utab_hbm, tailu_hbm, ulist, ucnt, u_emb, TRASH_U)
    run_sweep(itab_hbm, taili_hbm, ilist, icnt, i_emb, TRASH_I)


def _score_body(u_emb, i_emb, pos_out, neg_out,
                ubuf, pbuf, nbuf, psc, nsc, tp, tn, su, sp, sn):
    wid = lax.axis_index("s") * NC + lax.axis_index("c")
    base = wid * BPW

    tcol = lax.iota(jnp.int32, L) * (L + 1)

    for half in range(2):
        hbase = base + half * HB
        cu = pltpu.async_copy(u_emb.at[pl.ds(hbase, HB)], ubuf, su)
        cp = pltpu.async_copy(i_emb.at[pl.ds(hbase, HB)], pbuf, sp)
        cn = pltpu.async_copy(i_emb.at[pl.ds(B + hbase, HB)], nbuf, sn)
        cu.wait()
        cp.wait()
        cn.wait()

        def block(j, carry):
            i0 = j * L
            for k in range(L):
                i = i0 + k
                accp = jnp.zeros((L,), jnp.float32)
                accn = jnp.zeros((L,), jnp.float32)
                for q in range(D // L):
                    u = ubuf[i, pl.ds(q * L, L)]
                    accp = accp + u * pbuf[i, pl.ds(q * L, L)]
                    accn = accn + u * nbuf[i, pl.ds(q * L, L)]
                plsc.store_scatter(tp, [tcol + k], accp)
                plsc.store_scatter(tn, [tcol + k], accn)
            sp_ = jnp.zeros((L,), jnp.float32)
            sn_ = jnp.zeros((L,), jnp.float32)
            for l in range(L):
                sp_ = sp_ + tp[pl.ds(l * (L + 1), L)]
                sn_ = sn_ + tn[pl.ds(l * (L + 1), L)]
            psc[pl.ds(half * HB + i0, L)] = sp_
            nsc[pl.ds(half * HB + i0, L)] = sn_
            return carry

        lax.fori_loop(0, HB // L, block, 0, unroll=False)

    pltpu.sync_copy(psc, pos_out.at[pl.ds(base, BPW)])
    pltpu.sync_copy(nsc, neg_out.at[pl.ds(base, BPW)])


@jax.jit
def kernel(users, pos_items, neg_items, user_table, item_table):
    mesh = plsc.VectorSubcoreMesh(core_axis_name="c", subcore_axis_name="s",
                                  num_cores=NC, num_subcores=NS)
    sweep_k = pl.kernel(
        _sweep_body,
        out_type=(jax.ShapeDtypeStruct((B + 1, EMBW), jnp.float32),
                  jax.ShapeDtypeStruct((2 * B + 1, EMBW), jnp.float32)),
        mesh=mesh,
        scratch_types=[
            pltpu.VMEM((CB,), jnp.int32),
            pltpu.VMEM((UCAP,), jnp.int32),
            pltpu.VMEM((ICAP,), jnp.int32),
            pltpu.VMEM((D, SW), jnp.float32),
            pltpu.VMEM((D, D), jnp.float32),
            pltpu.VMEM((48,), jnp.int32),
            pltpu.VMEM((48,), jnp.int32),
            pltpu.VMEM((L * 65,), jnp.float32),
            pltpu.VMEM((L, EMBW), jnp.float32),
            pltpu.VMEM((L,), jnp.int32),
            pltpu.SemaphoreType.DMA,
        ],
        compiler_params=pltpu.CompilerParams(needs_layout_passes=False,
                                             use_tc_tiling_on_sc=True),
        name="bprmf_sweep",
    )
    score_k = pl.kernel(
        _score_body,
        out_type=(jax.ShapeDtypeStruct((B,), jnp.float32),
                  jax.ShapeDtypeStruct((B,), jnp.float32)),
        mesh=mesh,
        scratch_types=[
            pltpu.VMEM((HB, EMBW), jnp.float32),
            pltpu.VMEM((HB, EMBW), jnp.float32),
            pltpu.VMEM((HB, EMBW), jnp.float32),
            pltpu.VMEM((BPW,), jnp.float32),
            pltpu.VMEM((BPW,), jnp.float32),
            pltpu.VMEM((L * (L + 1),), jnp.float32),
            pltpu.VMEM((L * (L + 1),), jnp.float32),
            pltpu.SemaphoreType.DMA,
            pltpu.SemaphoreType.DMA,
            pltpu.SemaphoreType.DMA,
        ],
        compiler_params=pltpu.CompilerParams(needs_layout_passes=False,
                                             use_tc_tiling_on_sc=True),
        name="bprmf_score",
    )
    tail_u = lax.slice(user_table, (TAIL_LO, 0), (NROWS, D))
    tail_i = lax.slice(item_table, (TAIL_LO, 0), (NROWS, D))
    u_emb, i_emb = sweep_k(users, pos_items, neg_items,
                           user_table.T, item_table.T, tail_u, tail_i)
    return score_k(u_emb, i_emb)


# speedup vs baseline: 7.9197x; 1.7706x over previous
import functools

import jax
import jax.numpy as jnp
from jax import lax
from jax.experimental import pallas as pl
from jax.experimental.pallas import tpu as pltpu
from jax.experimental.pallas import tpu_sc as plsc

NROWS = 1000000
D = 64
B = 16384
NC = 2
NS = 16
NW = NC * NS
BPW = B // NW
L = 16
SW = 768
FULL_SLABS = 1302
WIN = 4096
TAIL_LO = FULL_SLABS * SW
TRASH_U = B
TRASH_I = 2 * B
UCAP = B + 16
ICAP = 2 * B + 16
CB = 8192
EMBW = 128
HB = BPW // 2


def _sweep_body(users_hbm, pos_hbm, neg_hbm, utab_hbm, itab_hbm,
                tailu_hbm, taili_hbm,
                u_emb, i_emb,
                cbuf, ulist, ilist, slab, tailblk,
                stg_r, stg_b, outbuf, outc, blist,
                osem):
    wid = lax.axis_index("s") * NC + lax.axis_index("c")
    nsl = jnp.where(wid < 22, 41, 40)
    lo = (wid * 40 + jnp.minimum(wid, 22)) * SW
    span = nsl * SW
    hi = jnp.where(wid == 31, NROWS, lo + span)

    iota = lax.iota(jnp.int32, L)
    tr65 = iota * 65

    def scan_array(arr, lst, b_off, cnt0):
        def chunk(ci, cnt):
            pltpu.sync_copy(arr.at[pl.ds(ci * CB, CB)], cbuf)

            def grp(g, c):
                r = cbuf[pl.ds(g * L, L)]
                m = (r >= lo) & (r < hi)
                b = ci * CB + g * L + b_off + iota
                h = (b << 15) | (r - lo)
                plsc.store_compressed(lst.at[pl.ds(c, L)], h, mask=m)
                return c + plsc.all_reduce_population_count(m)[0]

            return lax.fori_loop(0, CB // L, grp, cnt)

        return lax.fori_loop(0, B // CB, chunk, cnt0)

    ucnt = scan_array(users_hbm, ulist, 0, 0)
    ulist[pl.ds(ucnt, L)] = jnp.full((L,), 32767, jnp.int32)
    icnt = scan_array(pos_hbm, ilist, 0, 0)
    icnt = scan_array(neg_hbm, ilist, B, icnt)
    ilist[pl.ds(icnt, L)] = jnp.full((L,), 32767, jnp.int32)

    def flush16(rr, bb, emb, row_major_tail=False):
        pltpu.make_async_copy(outc, emb.at[blist], osem).wait()
        for d in range(D):
            dv = jnp.full((L,), d, jnp.int32)
            if row_major_tail:
                v = plsc.load_gather(tailblk, [rr, dv])
            else:
                v = plsc.load_gather(slab, [dv, rr])
            plsc.store_scatter(outbuf, [tr65 + d], v)
        for k in range(L):
            for q in range(D // L):
                outc[k, pl.ds(q * L, L)] = outbuf[pl.ds(k * 65 + q * L, L)]
        blist[pl.ds(0, L)] = bb
        pltpu.async_copy(outc, emb.at[blist], osem)

    def run_sweep(tab, tail_hbm, lst, lcnt, emb, trash):
        blist[pl.ds(0, L)] = jnp.full((L,), trash, jnp.int32)
        pltpu.async_copy(outc, emb.at[blist], osem)
        ngrp_all = (lcnt + L - 1) // L

        def scan_extract(s_lo, s_w, row_major_tail):
            def win(w, c):
                g0 = w * (WIN // L)
                gn = jnp.minimum(g0 + WIN // L, ngrp_all)

                def grp(g, ss):
                    h = lst[pl.ds(g * L, L)]
                    rl = h & 32767
                    m = (rl >= s_lo) & (rl < s_lo + s_w)
                    plsc.store_compressed(stg_r.at[pl.ds(ss, L)], rl - s_lo,
                                          mask=m)
                    plsc.store_compressed(stg_b.at[pl.ds(ss, L)], h >> 15,
                                          mask=m)
                    return ss + plsc.all_reduce_population_count(m)[0]

                ss = lax.fori_loop(g0, gn, grp, 0)

                @pl.when(ss > 0)
                def _():
                    stg_r[pl.ds(ss, L)] = jnp.zeros((L,), jnp.int32)
                    stg_b[pl.ds(ss, L)] = jnp.full((L,), trash, jnp.int32)

                    def fl(f, c2):
                        flush16(stg_r[pl.ds(f * L, L)],
                                stg_b[pl.ds(f * L, L)], emb, row_major_tail)
                        return c2

                    lax.fori_loop(0, (ss + L - 1) // L, fl, 0)

                return c

            lax.fori_loop(0, (lcnt + WIN - 1) // WIN, win, 0)

        def slab_iter(s, c):
            pltpu.sync_copy(tab.at[:, pl.ds(lo + s * SW, SW)], slab)
            scan_extract(s * SW, SW, False)
            return c

        lax.fori_loop(0, nsl, slab_iter, 0)

        @pl.when(wid == 31)
        def _():
            pltpu.sync_copy(tail_hbm, tailblk)
            scan_extract(nsl * SW, 64, True)

        pltpu.make_async_copy(outc, emb.at[blist], osem).wait()

    run_sweep(utab_hbm, tailu_hbm, ulist, ucnt, u_emb, TRASH_U)
    run_sweep(itab_hbm, taili_hbm, ilist, icnt, i_emb, TRASH_I)


def _score_body(u_emb, i_emb, pos_out, neg_out,
                ubuf, pbuf, nbuf, psc, nsc, tp, tn, su, sp, sn):
    wid = lax.axis_index("s") * NC + lax.axis_index("c")
    base = wid * BPW

    tcol = lax.iota(jnp.int32, L) * (L + 1)

    for half in range(2):
        hbase = base + half * HB
        cu = pltpu.async_copy(u_emb.at[pl.ds(hbase, HB)], ubuf, su)
        cp = pltpu.async_copy(i_emb.at[pl.ds(hbase, HB)], pbuf, sp)
        cn = pltpu.async_copy(i_emb.at[pl.ds(B + hbase, HB)], nbuf, sn)
        cu.wait()
        cp.wait()
        cn.wait()

        def block(j, carry):
            i0 = j * L
            for k in range(L):
                i = i0 + k
                accp = jnp.zeros((L,), jnp.float32)
                accn = jnp.zeros((L,), jnp.float32)
                for q in range(D // L):
                    u = ubuf[i, pl.ds(q * L, L)]
                    accp = accp + u * pbuf[i, pl.ds(q * L, L)]
                    accn = accn + u * nbuf[i, pl.ds(q * L, L)]
                plsc.store_scatter(tp, [tcol + k], accp)
                plsc.store_scatter(tn, [tcol + k], accn)
            sp_ = jnp.zeros((L,), jnp.float32)
            sn_ = jnp.zeros((L,), jnp.float32)
            for l in range(L):
                sp_ = sp_ + tp[pl.ds(l * (L + 1), L)]
                sn_ = sn_ + tn[pl.ds(l * (L + 1), L)]
            psc[pl.ds(half * HB + i0, L)] = sp_
            nsc[pl.ds(half * HB + i0, L)] = sn_
            return carry

        lax.fori_loop(0, HB // L, block, 0, unroll=False)

    pltpu.sync_copy(psc, pos_out.at[pl.ds(base, BPW)])
    pltpu.sync_copy(nsc, neg_out.at[pl.ds(base, BPW)])


@jax.jit
def kernel(users, pos_items, neg_items, user_table, item_table):
    mesh = plsc.VectorSubcoreMesh(core_axis_name="c", subcore_axis_name="s",
                                  num_cores=NC, num_subcores=NS)
    sweep_k = pl.kernel(
        _sweep_body,
        out_type=(jax.ShapeDtypeStruct((B + 1, EMBW), jnp.float32),
                  jax.ShapeDtypeStruct((2 * B + 1, EMBW), jnp.float32)),
        mesh=mesh,
        scratch_types=[
            pltpu.VMEM((CB,), jnp.int32),
            pltpu.VMEM((UCAP,), jnp.int32),
            pltpu.VMEM((ICAP,), jnp.int32),
            pltpu.VMEM((D, SW), jnp.float32),
            pltpu.VMEM((D, D), jnp.float32),
            pltpu.VMEM((WIN + L,), jnp.int32),
            pltpu.VMEM((WIN + L,), jnp.int32),
            pltpu.VMEM((L * 65,), jnp.float32),
            pltpu.VMEM((L, EMBW), jnp.float32),
            pltpu.VMEM((L,), jnp.int32),
            pltpu.SemaphoreType.DMA,
        ],
        compiler_params=pltpu.CompilerParams(needs_layout_passes=False,
                                             use_tc_tiling_on_sc=True),
        name="bprmf_sweep",
    )
    score_k = pl.kernel(
        _score_body,
        out_type=(jax.ShapeDtypeStruct((B,), jnp.float32),
                  jax.ShapeDtypeStruct((B,), jnp.float32)),
        mesh=mesh,
        scratch_types=[
            pltpu.VMEM((HB, EMBW), jnp.float32),
            pltpu.VMEM((HB, EMBW), jnp.float32),
            pltpu.VMEM((HB, EMBW), jnp.float32),
            pltpu.VMEM((BPW,), jnp.float32),
            pltpu.VMEM((BPW,), jnp.float32),
            pltpu.VMEM((L * (L + 1),), jnp.float32),
            pltpu.VMEM((L * (L + 1),), jnp.float32),
            pltpu.SemaphoreType.DMA,
            pltpu.SemaphoreType.DMA,
            pltpu.SemaphoreType.DMA,
        ],
        compiler_params=pltpu.CompilerParams(needs_layout_passes=False,
                                             use_tc_tiling_on_sc=True),
        name="bprmf_score",
    )
    tail_u = lax.slice(user_table, (TAIL_LO, 0), (NROWS, D))
    tail_i = lax.slice(item_table, (TAIL_LO, 0), (NROWS, D))
    u_emb, i_emb = sweep_k(users, pos_items, neg_items,
                           user_table.T, item_table.T, tail_u, tail_i)
    return score_k(u_emb, i_emb)
